# baseline (speedup 1.0000x reference)
import jax
import jax.numpy as jnp
from jax import lax
from jax.experimental import pallas as pl
from jax.experimental.pallas import tpu as pltpu
from jax.experimental.pallas import tpu_sc as plsc

B = 4096
D = 768
K1 = 1024
K2 = 256
NC = 2
NS = 16
NW = NC * NS
BPW = B // NW
L = 16

BLK = 256


def _make_sc_body(nu, ni):
    def body(ut, it, ub, ib, uid, iid, ug, ig, bsum,
             u0_v, u1_v, i0_v, i1_v, buf0, buf1, bu_v, bi_v,
             sem0, sem1):
        wid = lax.axis_index("s") * NC + lax.axis_index("c")
        base = wid * BPW
        sl = pl.ds(base, BPW)

        pltpu.sync_copy(uid.at[sl], u0_v)
        pltpu.sync_copy(iid.at[sl], i0_v)
        for j in range(BPW // L):
            s = pl.ds(j * L, L)
            u1_v[s] = u0_v[s] + nu
            i1_v[s] = i0_v[s] + ni

        bufs = (buf0, buf1)
        sems = (sem0, sem1)
        half = BPW // 2
        tasks = []
        for tab, idx, out, c in ((ut, u0_v, ug, 0), (ut, u1_v, ug, 1),
                                 (it, i0_v, ig, 0), (it, i1_v, ig, 1)):
            for h in (0, half):
                tasks.append((tab, idx.at[pl.ds(h, half)], out, c, h))
        nt = len(tasks)
        handles = [None] * nt
        handles[0] = pltpu.async_copy(tasks[0][0].at[tasks[0][1]], buf0, sem0)
        for t in range(nt):
            if t + 1 < nt:
                tab, idx, _, _, _ = tasks[t + 1]
                handles[t + 1] = pltpu.async_copy(
                    tab.at[idx], bufs[(t + 1) % 2], sems[(t + 1) % 2])
            handles[t].wait()
            _, _, out, c, h = tasks[t]
            pltpu.sync_copy(bufs[t % 2], out.at[c, pl.ds(base + h, half)])

        hu = pltpu.async_copy(ub.at[u0_v], bu_v, sem0)
        hi = pltpu.async_copy(ib.at[i0_v], bi_v, sem1)
        hu.wait()
        hi.wait()
        for j in range(BPW // L):
            s = pl.ds(j * L, L)
            bu_v[s] = bu_v[s] + bi_v[s]
        pltpu.sync_copy(bu_v, bsum.at[sl])

    return body


def _make_sc_gather(nu, ni):
    return pl.kernel(
        _make_sc_body(nu, ni),
        mesh=plsc.VectorSubcoreMesh(core_axis_name="c", subcore_axis_name="s"),
        out_type=[
            jax.ShapeDtypeStruct((2, B, D), jnp.float32),
            jax.ShapeDtypeStruct((2, B, D), jnp.float32),
            jax.ShapeDtypeStruct((B,), jnp.float32),
        ],
        scratch_types=[
            pltpu.VMEM((BPW,), jnp.int32),
            pltpu.VMEM((BPW,), jnp.int32),
            pltpu.VMEM((BPW,), jnp.int32),
            pltpu.VMEM((BPW,), jnp.int32),
            pltpu.VMEM((BPW // 2, D), jnp.float32),
            pltpu.VMEM((BPW // 2, D), jnp.float32),
            pltpu.VMEM((BPW,), jnp.float32),
            pltpu.VMEM((BPW,), jnp.float32),
            pltpu.SemaphoreType.DMA,
            pltpu.SemaphoreType.DMA,
        ],
    )


def _tc_ffn_body(ug_ref, ig_ref, wu1, bu1, wu2, bu2, wi1, bi1, wi2, bi2,
                 bsum_ref, out_ref):
    def two_layer(x, w1, b1, w2, b2):
        h = jnp.dot(x.astype(jnp.bfloat16), w1[...],
                    preferred_element_type=jnp.float32) + b1[...]
        h = jnp.maximum(h, 0.0)
        return jnp.dot(h.astype(jnp.bfloat16), w2[...],
                       preferred_element_type=jnp.float32) + b2[...]

    u = jnp.maximum(two_layer(ug_ref[0], wu1, bu1, wu2, bu2),
                    two_layer(ug_ref[1], wu1, bu1, wu2, bu2))
    v = jnp.maximum(two_layer(ig_ref[0], wi1, bi1, wi2, bi2),
                    two_layer(ig_ref[1], wi1, bi1, wi2, bi2))
    out_ref[...] = jnp.sum(u * v, axis=1) + bsum_ref[...]


_tc_ffn = pl.pallas_call(
    _tc_ffn_body,
    grid=(B // BLK,),
    in_specs=[
        pl.BlockSpec((2, BLK, D), lambda b: (0, b, 0)),
        pl.BlockSpec((2, BLK, D), lambda b: (0, b, 0)),
        pl.BlockSpec((D, K1), lambda b: (0, 0)),
        pl.BlockSpec((K1,), lambda b: (0,)),
        pl.BlockSpec((K1, K2), lambda b: (0, 0)),
        pl.BlockSpec((K2,), lambda b: (0,)),
        pl.BlockSpec((D, K1), lambda b: (0, 0)),
        pl.BlockSpec((K1,), lambda b: (0,)),
        pl.BlockSpec((K1, K2), lambda b: (0, 0)),
        pl.BlockSpec((K2,), lambda b: (0,)),
        pl.BlockSpec((BLK,), lambda b: (b,)),
    ],
    out_specs=pl.BlockSpec((BLK,), lambda b: (b,)),
    out_shape=jax.ShapeDtypeStruct((B,), jnp.float32),
)


def kernel(user_ids, item_ids, user_tables, item_tables, Wu1, bu1, Wu2, bu2,
           Wi1, bi1, Wi2, bi2, user_bias, item_bias):
    uids = user_ids[:, 0]
    iids = item_ids[:, 0]
    nu = user_tables.shape[1]
    ni = item_tables.shape[1]
    ut = user_tables.reshape(2 * nu, D)
    it = item_tables.reshape(2 * ni, D)
    ug, ig, bsum = _make_sc_gather(nu, ni)(ut, it, user_bias, item_bias,
                                           uids, iids)
    bf = jnp.bfloat16
    out = _tc_ffn(ug, ig, Wu1.astype(bf), bu1, Wu2.astype(bf), bu2,
                  Wi1.astype(bf), bi1, Wi2.astype(bf), bi2, bsum)
    return out[:, None]


# speedup vs baseline: 1.0815x; 1.0815x over previous
import jax
import jax.numpy as jnp
from jax import lax
from jax.experimental import pallas as pl
from jax.experimental.pallas import tpu as pltpu
from jax.experimental.pallas import tpu_sc as plsc

B = 4096
D = 768
K1 = 1024
K2 = 256
NC = 2
NS = 16
NW = NC * NS
L = 16

NSPLIT = 2
BS = B // NSPLIT
BLK = 512


def _make_sc_body(nu, ni, bpw):
    def body(ut, it, ub, ib, uid, iid, ug, ig, bsum,
             u0_v, u1_v, i0_v, i1_v, buf0, buf1, bu_v, bi_v,
             sem0, sem1):
        wid = lax.axis_index("s") * NC + lax.axis_index("c")
        base = wid * bpw
        sl = pl.ds(base, bpw)

        pltpu.sync_copy(uid.at[sl], u0_v)
        pltpu.sync_copy(iid.at[sl], i0_v)
        for j in range(bpw // L):
            s = pl.ds(j * L, L)
            u1_v[s] = u0_v[s] + nu
            i1_v[s] = i0_v[s] + ni

        bufs = (buf0, buf1)
        sems = (sem0, sem1)
        half = bpw // 2
        tasks = []
        for tab, idx, out, c in ((ut, u0_v, ug, 0), (ut, u1_v, ug, 1),
                                 (it, i0_v, ig, 0), (it, i1_v, ig, 1)):
            for h in (0, half):
                tasks.append((tab, idx.at[pl.ds(h, half)], out, c, h))
        nt = len(tasks)
        handles = [None] * nt
        handles[0] = pltpu.async_copy(tasks[0][0].at[tasks[0][1]], buf0, sem0)
        for t in range(nt):
            if t + 1 < nt:
                tab, idx, _, _, _ = tasks[t + 1]
                handles[t + 1] = pltpu.async_copy(
                    tab.at[idx], bufs[(t + 1) % 2], sems[(t + 1) % 2])
            handles[t].wait()
            _, _, out, c, h = tasks[t]
            pltpu.sync_copy(bufs[t % 2], out.at[c, pl.ds(base + h, half)])

        hu = pltpu.async_copy(ub.at[u0_v], bu_v, sem0)
        hi = pltpu.async_copy(ib.at[i0_v], bi_v, sem1)
        hu.wait()
        hi.wait()
        for j in range(bpw // L):
            s = pl.ds(j * L, L)
            bu_v[s] = bu_v[s] + bi_v[s]
        pltpu.sync_copy(bu_v, bsum.at[sl])

    return body


def _make_sc_gather(nu, ni, bs):
    bpw = bs // NW
    return pl.kernel(
        _make_sc_body(nu, ni, bpw),
        mesh=plsc.VectorSubcoreMesh(core_axis_name="c", subcore_axis_name="s"),
        out_type=[
            jax.ShapeDtypeStruct((2, bs, D), jnp.float32),
            jax.ShapeDtypeStruct((2, bs, D), jnp.float32),
            jax.ShapeDtypeStruct((bs,), jnp.float32),
        ],
        scratch_types=[
            pltpu.VMEM((bpw,), jnp.int32),
            pltpu.VMEM((bpw,), jnp.int32),
            pltpu.VMEM((bpw,), jnp.int32),
            pltpu.VMEM((bpw,), jnp.int32),
            pltpu.VMEM((bpw // 2, D), jnp.float32),
            pltpu.VMEM((bpw // 2, D), jnp.float32),
            pltpu.VMEM((bpw,), jnp.float32),
            pltpu.VMEM((bpw,), jnp.float32),
            pltpu.SemaphoreType.DMA,
            pltpu.SemaphoreType.DMA,
        ],
    )


def _tc_ffn_body(ug_ref, ig_ref, wu1, bu1, wu2, bu2, wi1, bi1, wi2, bi2,
                 bsum_ref, out_ref):
    def two_layer(x, w1, b1, w2, b2):
        h = jnp.dot(x.astype(jnp.bfloat16), w1[...],
                    preferred_element_type=jnp.float32) + b1[...]
        h = jnp.maximum(h, 0.0)
        return jnp.dot(h.astype(jnp.bfloat16), w2[...],
                       preferred_element_type=jnp.float32) + b2[...]

    u = jnp.maximum(two_layer(ug_ref[0], wu1, bu1, wu2, bu2),
                    two_layer(ug_ref[1], wu1, bu1, wu2, bu2))
    v = jnp.maximum(two_layer(ig_ref[0], wi1, bi1, wi2, bi2),
                    two_layer(ig_ref[1], wi1, bi1, wi2, bi2))
    out_ref[...] = jnp.sum(u * v, axis=1) + bsum_ref[...]


_tc_ffn = pl.pallas_call(
    _tc_ffn_body,
    grid=(BS // BLK,),
    in_specs=[
        pl.BlockSpec((2, BLK, D), lambda b: (0, b, 0)),
        pl.BlockSpec((2, BLK, D), lambda b: (0, b, 0)),
        pl.BlockSpec((D, K1), lambda b: (0, 0)),
        pl.BlockSpec((K1,), lambda b: (0,)),
        pl.BlockSpec((K1, K2), lambda b: (0, 0)),
        pl.BlockSpec((K2,), lambda b: (0,)),
        pl.BlockSpec((D, K1), lambda b: (0, 0)),
        pl.BlockSpec((K1,), lambda b: (0,)),
        pl.BlockSpec((K1, K2), lambda b: (0, 0)),
        pl.BlockSpec((K2,), lambda b: (0,)),
        pl.BlockSpec((BLK,), lambda b: (b,)),
    ],
    out_specs=pl.BlockSpec((BLK,), lambda b: (b,)),
    out_shape=jax.ShapeDtypeStruct((BS,), jnp.float32),
)


def kernel(user_ids, item_ids, user_tables, item_tables, Wu1, bu1, Wu2, bu2,
           Wi1, bi1, Wi2, bi2, user_bias, item_bias):
    uids = user_ids[:, 0]
    iids = item_ids[:, 0]
    nu = user_tables.shape[1]
    ni = item_tables.shape[1]
    ut = user_tables.reshape(2 * nu, D)
    it = item_tables.reshape(2 * ni, D)
    bf = jnp.bfloat16
    w = (Wu1.astype(bf), bu1, Wu2.astype(bf), bu2,
         Wi1.astype(bf), bi1, Wi2.astype(bf), bi2)
    sc = _make_sc_gather(nu, ni, BS)
    outs = []
    for s in range(NSPLIT):
        lo = s * BS
        ug, ig, bsum = sc(ut, it, user_bias, item_bias,
                          lax.dynamic_slice(uids, (lo,), (BS,)),
                          lax.dynamic_slice(iids, (lo,), (BS,)))
        outs.append(_tc_ffn(ug, ig, *w, bsum))
    out = jnp.concatenate(outs)
    return out[:, None]


# speedup vs baseline: 1.0952x; 1.0127x over previous
import jax
import jax.numpy as jnp
from jax import lax
from jax.experimental import pallas as pl
from jax.experimental.pallas import tpu as pltpu
from jax.experimental.pallas import tpu_sc as plsc

B = 4096
D = 768
K1 = 1024
K2 = 256
NC = 2
NS = 16
NW = NC * NS
L = 16

NSPLIT = 2
BS = B // NSPLIT
BLK = 512


def _make_sc_body(nu, ni, bpw, lo):
    def body(ut, it, ub, ib, uid, iid, ug, ig, bsum,
             u0_v, u1_v, i0_v, i1_v, buf0, buf1, bu_v, bi_v,
             sem0, sem1, semb0, semb1):
        wid = lax.axis_index("s") * NC + lax.axis_index("c")
        base = wid * bpw
        sl = pl.ds(base, bpw)

        pltpu.sync_copy(uid.at[pl.ds(lo + base, bpw)], u0_v)
        pltpu.sync_copy(iid.at[pl.ds(lo + base, bpw)], i0_v)
        hu = pltpu.async_copy(ub.at[u0_v], bu_v, semb0)
        hi = pltpu.async_copy(ib.at[i0_v], bi_v, semb1)
        for j in range(bpw // L):
            s = pl.ds(j * L, L)
            u1_v[s] = u0_v[s] + nu
            i1_v[s] = i0_v[s] + ni

        bufs = (buf0, buf1)
        sems = (sem0, sem1)
        half = bpw // 2
        tasks = []
        for tab, idx, out, c in ((ut, u0_v, ug, 0), (ut, u1_v, ug, 1),
                                 (it, i0_v, ig, 0), (it, i1_v, ig, 1)):
            for h in (0, half):
                tasks.append((tab, idx.at[pl.ds(h, half)], out, c, h))
        nt = len(tasks)
        handles = [None] * nt
        handles[0] = pltpu.async_copy(tasks[0][0].at[tasks[0][1]], buf0, sem0)
        for t in range(nt):
            if t + 1 < nt:
                tab, idx, _, _, _ = tasks[t + 1]
                handles[t + 1] = pltpu.async_copy(
                    tab.at[idx], bufs[(t + 1) % 2], sems[(t + 1) % 2])
            handles[t].wait()
            _, _, out, c, h = tasks[t]
            pltpu.sync_copy(bufs[t % 2], out.at[c, pl.ds(base + h, half)])

        hu.wait()
        hi.wait()
        for j in range(bpw // L):
            s = pl.ds(j * L, L)
            bu_v[s] = bu_v[s] + bi_v[s]
        pltpu.sync_copy(bu_v, bsum.at[sl])

    return body


def _make_sc_gather(nu, ni, bs, lo):
    bpw = bs // NW
    return pl.kernel(
        _make_sc_body(nu, ni, bpw, lo),
        mesh=plsc.VectorSubcoreMesh(core_axis_name="c", subcore_axis_name="s"),
        out_type=[
            jax.ShapeDtypeStruct((2, bs, D), jnp.float32),
            jax.ShapeDtypeStruct((2, bs, D), jnp.float32),
            jax.ShapeDtypeStruct((bs,), jnp.float32),
        ],
        scratch_types=[
            pltpu.VMEM((bpw,), jnp.int32),
            pltpu.VMEM((bpw,), jnp.int32),
            pltpu.VMEM((bpw,), jnp.int32),
            pltpu.VMEM((bpw,), jnp.int32),
            pltpu.VMEM((bpw // 2, D), jnp.float32),
            pltpu.VMEM((bpw // 2, D), jnp.float32),
            pltpu.VMEM((bpw,), jnp.float32),
            pltpu.VMEM((bpw,), jnp.float32),
            pltpu.SemaphoreType.DMA,
            pltpu.SemaphoreType.DMA,
            pltpu.SemaphoreType.DMA,
            pltpu.SemaphoreType.DMA,
        ],
    )


def _tc_ffn_body(ug_ref, ig_ref, wu1, bu1, wu2, bu2, wi1, bi1, wi2, bi2,
                 bsum_ref, out_ref):
    def two_layer(x, w1, b1, w2, b2):
        h = jnp.dot(x.astype(jnp.bfloat16), w1[...],
                    preferred_element_type=jnp.float32) + b1[...]
        h = jnp.maximum(h, 0.0)
        return jnp.dot(h.astype(jnp.bfloat16), w2[...],
                       preferred_element_type=jnp.float32) + b2[...]

    u = jnp.maximum(two_layer(ug_ref[0], wu1, bu1, wu2, bu2),
                    two_layer(ug_ref[1], wu1, bu1, wu2, bu2))
    v = jnp.maximum(two_layer(ig_ref[0], wi1, bi1, wi2, bi2),
                    two_layer(ig_ref[1], wi1, bi1, wi2, bi2))
    out_ref[...] = jnp.sum(u * v, axis=1) + bsum_ref[...]


_tc_ffn = pl.pallas_call(
    _tc_ffn_body,
    grid=(BS // BLK,),
    in_specs=[
        pl.BlockSpec((2, BLK, D), lambda b: (0, b, 0)),
        pl.BlockSpec((2, BLK, D), lambda b: (0, b, 0)),
        pl.BlockSpec((D, K1), lambda b: (0, 0)),
        pl.BlockSpec((K1,), lambda b: (0,)),
        pl.BlockSpec((K1, K2), lambda b: (0, 0)),
        pl.BlockSpec((K2,), lambda b: (0,)),
        pl.BlockSpec((D, K1), lambda b: (0, 0)),
        pl.BlockSpec((K1,), lambda b: (0,)),
        pl.BlockSpec((K1, K2), lambda b: (0, 0)),
        pl.BlockSpec((K2,), lambda b: (0,)),
        pl.BlockSpec((BLK,), lambda b: (b,)),
    ],
    out_specs=pl.BlockSpec((BLK,), lambda b: (b,)),
    out_shape=jax.ShapeDtypeStruct((BS,), jnp.float32),
)


def kernel(user_ids, item_ids, user_tables, item_tables, Wu1, bu1, Wu2, bu2,
           Wi1, bi1, Wi2, bi2, user_bias, item_bias):
    uids = user_ids[:, 0]
    iids = item_ids[:, 0]
    nu = user_tables.shape[1]
    ni = item_tables.shape[1]
    ut = user_tables.reshape(2 * nu, D)
    it = item_tables.reshape(2 * ni, D)
    bf = jnp.bfloat16
    w = (Wu1.astype(bf), bu1, Wu2.astype(bf), bu2,
         Wi1.astype(bf), bi1, Wi2.astype(bf), bi2)
    outs = []
    for s in range(NSPLIT):
        sc = _make_sc_gather(nu, ni, BS, s * BS)
        ug, ig, bsum = sc(ut, it, user_bias, item_bias, uids, iids)
        outs.append(_tc_ffn(ug, ig, *w, bsum))
    out = jnp.concatenate(outs)
    return out[:, None]
